# Initial kernel scaffold; baseline (speedup 1.0000x reference)
import jax
import jax.numpy as jnp
from jax.experimental import pallas as pl


def kernel(f_atoms, f_bonds, a2b, b2a, b2revb, undirected_b2a, w_atom, mlp_W1, mlp_b1, mlp_W2, mlp_b2, wb_W1, wb_b1, wb_W2, wb_b2, bn_g, bn_b):
    raise NotImplementedError("write your pallas kernel here")


# speedup vs baseline: 2.6308x; 2.6308x over previous
import functools

import jax
import jax.numpy as jnp
from jax import lax
from jax.experimental import pallas as pl
from jax.experimental.pallas import tpu as pltpu
from jax.experimental.pallas import tpu_sc as plsc

_EPS = 1e-5
_NC = 2
_NS = 16
_CHUNK = 128


def _h0_body(x_ref, wt_ref, o_ref):
    o_ref[...] = jnp.dot(x_ref[...], wt_ref[...],
                         preferred_element_type=jnp.float32)


def _edge_body(fb_ref, w1_ref, b1_ref, w2_ref, b2_ref, o_ref):
    t = jnp.dot(fb_ref[...], w1_ref[...],
                preferred_element_type=jnp.float32) + b1_ref[...]
    t = jnp.maximum(t, 0.0)
    o_ref[...] = jnp.dot(t, w2_ref[...],
                         preferred_element_type=jnp.float32) + b2_ref[...]


def _node_body(p0_ref, p1_ref, h_ref, w1c_ref, b1e_ref, w2e_ref, b2e_ref,
               w1t_ref, b1m_ref, w2t_ref, b2m_ref, g_ref, bb_ref, o_ref,
               *, relu_out):
    ee_loop = jnp.maximum(w1c_ref[...] + b1e_ref[...], 0.0)
    ee_loop = jnp.dot(ee_loop, w2e_ref[...],
                      preferred_element_type=jnp.float32) + b2e_ref[...]
    aggr = p0_ref[...] + p1_ref[...] + h_ref[...] + ee_loop
    t = jnp.dot(aggr, w1t_ref[...],
                preferred_element_type=jnp.float32) + b1m_ref[...]
    t = jnp.maximum(t, 0.0)
    hh = jnp.dot(t, w2t_ref[...],
                 preferred_element_type=jnp.float32) + b2m_ref[...]
    scale = 1.0 / jnp.sqrt(1.0 + _EPS)
    hh = hh * (g_ref[...] * scale) + bb_ref[...]
    if relu_out:
        hh = jnp.maximum(hh, 0.0)
    o_ref[...] = hh


def _full_spec(shape):
    nd = len(shape)
    return pl.BlockSpec(shape, lambda i: (0,) * nd)


def _h0(f_atoms, w_atom_t, bn=2000):
    n, d = f_atoms.shape
    return pl.pallas_call(
        _h0_body,
        grid=(n // bn,),
        in_specs=[pl.BlockSpec((bn, d), lambda i: (i, 0)),
                  _full_spec(w_atom_t.shape)],
        out_specs=pl.BlockSpec((bn, d), lambda i: (i, 0)),
        out_shape=jax.ShapeDtypeStruct((n, d), jnp.float32),
    )(f_atoms, w_atom_t)


def _edge_mlp(f_bonds, w1e, b1e, w2e, b2e, be=2000):
    e, k = f_bonds.shape
    d = w2e.shape[1]
    return pl.pallas_call(
        _edge_body,
        grid=(e // be,),
        in_specs=[pl.BlockSpec((be, k), lambda i: (i, 0)),
                  _full_spec(w1e.shape), _full_spec(b1e.shape),
                  _full_spec(w2e.shape), _full_spec(b2e.shape)],
        out_specs=pl.BlockSpec((be, d), lambda i: (i, 0)),
        out_shape=jax.ShapeDtypeStruct((e, d), jnp.float32),
    )(f_bonds, w1e, b1e, w2e, b2e)


def _node_update(p0, p1, h, w1c, b1e, w2e, b2e, w1t, b1m, w2t, b2m, g, bb,
                 relu_out, bn=2000):
    n, d = h.shape
    body = functools.partial(_node_body, relu_out=relu_out)
    row = pl.BlockSpec((bn, d), lambda i: (i, 0))
    return pl.pallas_call(
        body,
        grid=(n // bn,),
        in_specs=[row, row, row,
                  _full_spec(w1c.shape), _full_spec(b1e.shape),
                  _full_spec(w2e.shape), _full_spec(b2e.shape),
                  _full_spec(w1t.shape), _full_spec(b1m.shape),
                  _full_spec(w2t.shape), _full_spec(b2m.shape),
                  _full_spec(g.shape), _full_spec(bb.shape)],
        out_specs=row,
        out_shape=jax.ShapeDtypeStruct((n, d), jnp.float32),
    )(p0, p1, h, w1c, b1e, w2e, b2e, w1t, b1m, w2t, b2m, g, bb)


def _make_sc_aggregate(n, e, d, npad):
    c = _CHUNK
    epw = e // (_NC * _NS)
    nfull = epw // c
    rem = epw - nfull * c
    rps = npad // _NS
    mesh = plsc.VectorSubcoreMesh(core_axis_name="c", subcore_axis_name="s")

    scratch = [
        pltpu.VMEM((c,), jnp.int32),
        pltpu.VMEM((c,), jnp.int32),
        pltpu.VMEM((c, d), jnp.float32),
        pltpu.VMEM((c, d), jnp.float32),
        pltpu.VMEM_SHARED((npad, d), jnp.float32),
        pltpu.SemaphoreType.DMA,
    ]
    if rem:
        scratch += [pltpu.VMEM((rem,), jnp.int32),
                    pltpu.VMEM((rem,), jnp.int32)]

    @functools.partial(
        pl.kernel,
        mesh=mesh,
        out_type=jax.ShapeDtypeStruct((_NC, npad, d), jnp.float32),
        scratch_types=scratch,
    )
    def sc(h_hbm, ee_hbm, src_hbm, dst_hbm, z_hbm, out_hbm,
           src_v, dst_v, hrows_v, ee_v, acc, sem, *rem_scratch):
        cid = lax.axis_index("c")
        sid = lax.axis_index("s")
        wid = cid * _NS + sid
        base0 = wid * epw

        pltpu.sync_copy(z_hbm.at[pl.ds(sid * rps, rps)],
                        acc.at[pl.ds(sid * rps, rps)])
        plsc.subcore_barrier()

        def chunk_step(base, src_i, dst_i, hrows_i, ee_i, width):
            pltpu.sync_copy(src_hbm.at[pl.ds(base, width)], src_i)
            pltpu.sync_copy(dst_hbm.at[pl.ds(base, width)], dst_i)
            pltpu.async_copy(h_hbm.at[src_i], hrows_i, sem).wait()
            pltpu.sync_copy(ee_hbm.at[pl.ds(base, width)], ee_i)
            pltpu.sync_copy(hrows_i, acc.at[dst_i], add=True)
            pltpu.sync_copy(ee_i, acc.at[dst_i], add=True)

        def body(t, carry):
            chunk_step(base0 + t * c, src_v, dst_v, hrows_v, ee_v, c)
            return carry

        lax.fori_loop(0, nfull, body, 0)

        if rem:
            src_r, dst_r = rem_scratch
            chunk_step(base0 + nfull * c, src_r, dst_r,
                       hrows_v.at[pl.ds(0, rem)], ee_v.at[pl.ds(0, rem)],
                       rem)

        plsc.subcore_barrier()
        pltpu.sync_copy(acc.at[pl.ds(sid * rps, rps)],
                        out_hbm.at[cid].at[pl.ds(sid * rps, rps)])

    return sc


def _aggregate_partials(h, ee, src, dst, zeros_nd):
    n, d = h.shape
    e = src.shape[0]
    npad = zeros_nd.shape[0]
    return _make_sc_aggregate(n, e, d, npad)(h, ee, src, dst, zeros_nd)


def kernel(f_atoms, f_bonds, a2b, b2a, b2revb, undirected_b2a, w_atom,
           mlp_W1, mlp_b1, mlp_W2, mlp_b2, wb_W1, wb_b1, wb_W2, wb_b2,
           bn_g, bn_b):
    n, d = f_atoms.shape
    e = b2a.shape[0]
    bf1 = f_bonds.shape[1]
    depth = mlp_W1.shape[0]

    mask = a2b > 0
    rows_full = jnp.broadcast_to(
        jnp.arange(a2b.shape[0], dtype=b2a.dtype)[:, None], a2b.shape)
    safe_ids = jnp.where(mask, a2b, e)
    to_ext = jnp.zeros((e + 1,), dtype=b2a.dtype)
    to_ext = to_ext.at[safe_ids.ravel()].set(rows_full.ravel())
    dst = to_ext[:e]
    src = b2a

    npad = ((n + 8 * _NS - 1) // (8 * _NS)) * (8 * _NS)
    zeros_nd = jnp.zeros((npad, d), jnp.float32)

    h = _h0(f_atoms, w_atom.T)

    for l in range(depth):
        w1e = wb_W1[l][:, :bf1].T
        b1e = wb_b1[l].reshape(1, d)
        w2e = wb_W2[l].T
        b2e = wb_b2[l].reshape(1, d)
        w1c = wb_W1[l][:, bf1].reshape(1, d)

        ee = _edge_mlp(f_bonds, w1e, b1e, w2e, b2e)
        partials = _aggregate_partials(h, ee, src, dst, zeros_nd)

        w1t = mlp_W1[l].T
        b1m = mlp_b1[l].reshape(1, -1)
        w2t = mlp_W2[l].T
        b2m = mlp_b2[l].reshape(1, d)
        g = bn_g[l].reshape(1, d)
        bb = bn_b[l].reshape(1, d)

        h = _node_update(partials[0], partials[1], h,
                         w1c, b1e, w2e, b2e,
                         w1t, b1m, w2t, b2m, g, bb,
                         relu_out=(l < depth - 1))

    return h


# speedup vs baseline: 4.7444x; 1.8034x over previous
import functools

import jax
import jax.numpy as jnp
from jax import lax
from jax.experimental import pallas as pl
from jax.experimental.pallas import tpu as pltpu
from jax.experimental.pallas import tpu_sc as plsc

_EPS = 1e-5
_NC = 2
_NS = 16
_CHUNK = 128


def _h0_body(x_ref, wt_ref, o_ref):
    o_ref[...] = jnp.dot(x_ref[...], wt_ref[...],
                         preferred_element_type=jnp.float32)


def _edge_body(fb_ref, w1_ref, b1_ref, w2_ref, b2_ref, o_ref):
    t = jnp.dot(fb_ref[...], w1_ref[...],
                preferred_element_type=jnp.float32) + b1_ref[...]
    t = jnp.maximum(t, 0.0)
    o_ref[...] = jnp.dot(t, w2_ref[...],
                         preferred_element_type=jnp.float32) + b2_ref[...]


def _node_body(p0_ref, p1_ref, h_ref, w1c_ref, b1e_ref, w2e_ref, b2e_ref,
               w1t_ref, b1m_ref, w2t_ref, b2m_ref, g_ref, bb_ref, o_ref,
               *, relu_out):
    ee_loop = jnp.maximum(w1c_ref[...] + b1e_ref[...], 0.0)
    ee_loop = jnp.dot(ee_loop, w2e_ref[...],
                      preferred_element_type=jnp.float32) + b2e_ref[...]
    aggr = p0_ref[...] + p1_ref[...] + h_ref[...] + ee_loop
    t = jnp.dot(aggr, w1t_ref[...],
                preferred_element_type=jnp.float32) + b1m_ref[...]
    t = jnp.maximum(t, 0.0)
    hh = jnp.dot(t, w2t_ref[...],
                 preferred_element_type=jnp.float32) + b2m_ref[...]
    scale = 1.0 / jnp.sqrt(1.0 + _EPS)
    hh = hh * (g_ref[...] * scale) + bb_ref[...]
    if relu_out:
        hh = jnp.maximum(hh, 0.0)
    o_ref[...] = hh


def _full_spec(shape):
    nd = len(shape)
    return pl.BlockSpec(shape, lambda i: (0,) * nd)


def _h0(f_atoms, w_atom_t, bn=2000):
    n, d = f_atoms.shape
    return pl.pallas_call(
        _h0_body,
        grid=(n // bn,),
        in_specs=[pl.BlockSpec((bn, d), lambda i: (i, 0)),
                  _full_spec(w_atom_t.shape)],
        out_specs=pl.BlockSpec((bn, d), lambda i: (i, 0)),
        out_shape=jax.ShapeDtypeStruct((n, d), jnp.float32),
    )(f_atoms, w_atom_t)


def _edge_mlp(f_bonds, w1e, b1e, w2e, b2e, be=2000):
    e, k = f_bonds.shape
    d = w2e.shape[1]
    return pl.pallas_call(
        _edge_body,
        grid=(e // be,),
        in_specs=[pl.BlockSpec((be, k), lambda i: (i, 0)),
                  _full_spec(w1e.shape), _full_spec(b1e.shape),
                  _full_spec(w2e.shape), _full_spec(b2e.shape)],
        out_specs=pl.BlockSpec((be, d), lambda i: (i, 0)),
        out_shape=jax.ShapeDtypeStruct((e, d), jnp.float32),
    )(f_bonds, w1e, b1e, w2e, b2e)


def _node_update(p0, p1, h, w1c, b1e, w2e, b2e, w1t, b1m, w2t, b2m, g, bb,
                 relu_out, bn=2000):
    n, d = h.shape
    body = functools.partial(_node_body, relu_out=relu_out)
    row = pl.BlockSpec((bn, d), lambda i: (i, 0))
    return pl.pallas_call(
        body,
        grid=(n // bn,),
        in_specs=[row, row, row,
                  _full_spec(w1c.shape), _full_spec(b1e.shape),
                  _full_spec(w2e.shape), _full_spec(b2e.shape),
                  _full_spec(w1t.shape), _full_spec(b1m.shape),
                  _full_spec(w2t.shape), _full_spec(b2m.shape),
                  _full_spec(g.shape), _full_spec(bb.shape)],
        out_specs=row,
        out_shape=jax.ShapeDtypeStruct((n, d), jnp.float32),
    )(p0, p1, h, w1c, b1e, w2e, b2e, w1t, b1m, w2t, b2m, g, bb)


def _make_sc_aggregate(n, e, d, npad):
    c = _CHUNK
    epw = e // (_NC * _NS)
    nfull = epw // c
    rem = epw - nfull * c
    rps = npad // _NS
    mesh = plsc.VectorSubcoreMesh(core_axis_name="c", subcore_axis_name="s")

    scratch = [
        pltpu.VMEM((c,), jnp.int32),
        pltpu.VMEM((c,), jnp.int32),
        pltpu.VMEM((c, d), jnp.float32),
        pltpu.VMEM((c, d), jnp.float32),
        pltpu.VMEM_SHARED((npad, d), jnp.float32),
        pltpu.SemaphoreType.DMA,
    ]
    if rem:
        scratch += [pltpu.VMEM((rem,), jnp.int32),
                    pltpu.VMEM((rem,), jnp.int32)]

    @functools.partial(
        pl.kernel,
        mesh=mesh,
        out_type=jax.ShapeDtypeStruct((_NC, npad, d), jnp.float32),
        scratch_types=scratch,
    )
    def sc(h_hbm, ee_hbm, src_hbm, dst_hbm, z_hbm, out_hbm,
           src_v, dst_v, hrows_v, ee_v, acc, sem, *rem_scratch):
        cid = lax.axis_index("c")
        sid = lax.axis_index("s")
        wid = cid * _NS + sid
        base0 = wid * epw

        pltpu.sync_copy(z_hbm.at[pl.ds(sid * rps, rps)],
                        acc.at[pl.ds(sid * rps, rps)])
        plsc.subcore_barrier()

        def chunk_step(base, src_i, dst_i, hrows_i, ee_i, width):
            pltpu.sync_copy(src_hbm.at[pl.ds(base, width)], src_i)
            pltpu.sync_copy(dst_hbm.at[pl.ds(base, width)], dst_i)
            pltpu.async_copy(h_hbm.at[src_i], hrows_i, sem).wait()
            pltpu.sync_copy(ee_hbm.at[pl.ds(base, width)], ee_i)
            pltpu.sync_copy(hrows_i, acc.at[dst_i], add=True)
            pltpu.sync_copy(ee_i, acc.at[dst_i], add=True)

        def body(t, carry):
            chunk_step(base0 + t * c, src_v, dst_v, hrows_v, ee_v, c)
            return carry

        lax.fori_loop(0, nfull, body, 0)

        if rem:
            src_r, dst_r = rem_scratch
            chunk_step(base0 + nfull * c, src_r, dst_r,
                       hrows_v.at[pl.ds(0, rem)], ee_v.at[pl.ds(0, rem)],
                       rem)

        plsc.subcore_barrier()
        pltpu.sync_copy(acc.at[pl.ds(sid * rps, rps)],
                        out_hbm.at[cid].at[pl.ds(sid * rps, rps)])

    return sc


def _aggregate_partials(h, ee, src, dst, zeros_nd):
    n, d = h.shape
    e = src.shape[0]
    npad = zeros_nd.shape[0]
    return _make_sc_aggregate(n, e, d, npad)(h, ee, src, dst, zeros_nd)


def _make_sc_to(n_slots, e, maxnb):
    nw = _NC * _NS
    ids_pw = e // nw
    chunk = 2000
    nchunk = n_slots // chunk
    tail = n_slots - nchunk * chunk
    mesh = plsc.VectorSubcoreMesh(core_axis_name="c", subcore_axis_name="s")

    @functools.partial(
        pl.kernel,
        mesh=mesh,
        out_type=jax.ShapeDtypeStruct((e,), jnp.int32),
        scratch_types=[
            pltpu.VMEM((chunk,), jnp.int32),
            pltpu.VMEM((ids_pw,), jnp.int32),
        ],
        compiler_params=pltpu.CompilerParams(needs_layout_passes=False),
    )
    def sc(a2b_hbm, zi_hbm, to_hbm, buf_v, tbl_v):
        cid = lax.axis_index("c")
        sid = lax.axis_index("s")
        wid = cid * _NS + sid
        lo = wid * ids_pw
        lo1 = jnp.maximum(lo, 1)
        hi = lo + ids_pw
        lane = lax.iota(jnp.int32, 16)
        ltrue = lane >= jnp.minimum(sid, 0)

        pltpu.sync_copy(zi_hbm, tbl_v)

        def scan_block(p0, nvec):
            def vbody(k, carry):
                off = k * 16 + lane
                v = plsc.load_gather(buf_v, [off], mask=ltrue)
                mask = (v >= lo1) & (v < hi)
                _, last = plsc.scan_count(v, mask)
                row = lax.div(p0 + off, jnp.int32(maxnb))
                plsc.store_scatter(tbl_v, [v - lo], row, mask=last)
                return carry
            lax.fori_loop(0, nvec, vbody, 0)

        def cbody(t, carry):
            pltpu.sync_copy(a2b_hbm.at[pl.ds(t * chunk, chunk)], buf_v)
            scan_block(t * chunk, chunk // 16)
            return carry
        lax.fori_loop(0, nchunk, cbody, 0)

        if tail:
            pltpu.sync_copy(a2b_hbm.at[pl.ds(nchunk * chunk, tail)],
                            buf_v.at[pl.ds(0, tail)])
            scan_block(nchunk * chunk, tail // 16)

        pltpu.sync_copy(tbl_v, to_hbm.at[pl.ds(lo, ids_pw)])

    return sc


def kernel(f_atoms, f_bonds, a2b, b2a, b2revb, undirected_b2a, w_atom,
           mlp_W1, mlp_b1, mlp_W2, mlp_b2, wb_W1, wb_b1, wb_W2, wb_b2,
           bn_g, bn_b):
    n, d = f_atoms.shape
    e = b2a.shape[0]
    bf1 = f_bonds.shape[1]
    depth = mlp_W1.shape[0]

    maxnb = a2b.shape[1]
    a2b_flat = a2b.reshape(-1)
    zi = jnp.zeros((e // (_NC * _NS),), jnp.int32)
    dst = _make_sc_to(a2b_flat.shape[0], e, maxnb)(a2b_flat, zi)
    src = b2a

    npad = ((n + 8 * _NS - 1) // (8 * _NS)) * (8 * _NS)
    zeros_nd = jnp.zeros((npad, d), jnp.float32)

    h = _h0(f_atoms, w_atom.T)

    for l in range(depth):
        w1e = wb_W1[l][:, :bf1].T
        b1e = wb_b1[l].reshape(1, d)
        w2e = wb_W2[l].T
        b2e = wb_b2[l].reshape(1, d)
        w1c = wb_W1[l][:, bf1].reshape(1, d)

        ee = _edge_mlp(f_bonds, w1e, b1e, w2e, b2e)
        partials = _aggregate_partials(h, ee, src, dst, zeros_nd)

        w1t = mlp_W1[l].T
        b1m = mlp_b1[l].reshape(1, -1)
        w2t = mlp_W2[l].T
        b2m = mlp_b2[l].reshape(1, d)
        g = bn_g[l].reshape(1, d)
        bb = bn_b[l].reshape(1, d)

        h = _node_update(partials[0], partials[1], h,
                         w1c, b1e, w2e, b2e,
                         w1t, b1m, w2t, b2m, g, bb,
                         relu_out=(l < depth - 1))

    return h


# speedup vs baseline: 6.5723x; 1.3853x over previous
import functools

import jax
import jax.numpy as jnp
from jax import lax
from jax.experimental import pallas as pl
from jax.experimental.pallas import tpu as pltpu
from jax.experimental.pallas import tpu_sc as plsc

_EPS = 1e-5
_NC = 2
_NS = 16
_CHUNK = 80


def _h0_body(x_ref, wt_ref, o_ref):
    o_ref[...] = jnp.dot(x_ref[...], wt_ref[...],
                         preferred_element_type=jnp.float32)


def _edge_body(fb_ref, w1_ref, b1_ref, w2_ref, b2_ref, o_ref):
    t = jnp.dot(fb_ref[...], w1_ref[...],
                preferred_element_type=jnp.float32) + b1_ref[...]
    t = jnp.maximum(t, 0.0)
    o_ref[...] = jnp.dot(t, w2_ref[...],
                         preferred_element_type=jnp.float32) + b2_ref[...]


def _node_body(p0_ref, p1_ref, h_ref, w1c_ref, b1e_ref, w2e_ref, b2e_ref,
               w1t_ref, b1m_ref, w2t_ref, b2m_ref, g_ref, bb_ref, o_ref,
               *, relu_out):
    ee_loop = jnp.maximum(w1c_ref[...] + b1e_ref[...], 0.0)
    ee_loop = jnp.dot(ee_loop, w2e_ref[...],
                      preferred_element_type=jnp.float32) + b2e_ref[...]
    aggr = p0_ref[...] + p1_ref[...] + h_ref[...] + ee_loop
    t = jnp.dot(aggr, w1t_ref[...],
                preferred_element_type=jnp.float32) + b1m_ref[...]
    t = jnp.maximum(t, 0.0)
    hh = jnp.dot(t, w2t_ref[...],
                 preferred_element_type=jnp.float32) + b2m_ref[...]
    scale = 1.0 / jnp.sqrt(1.0 + _EPS)
    hh = hh * (g_ref[...] * scale) + bb_ref[...]
    if relu_out:
        hh = jnp.maximum(hh, 0.0)
    o_ref[...] = hh


def _full_spec(shape):
    nd = len(shape)
    return pl.BlockSpec(shape, lambda i: (0,) * nd)


def _h0(f_atoms, w_atom_t, bn=2000):
    n, d = f_atoms.shape
    return pl.pallas_call(
        _h0_body,
        grid=(n // bn,),
        in_specs=[pl.BlockSpec((bn, d), lambda i: (i, 0)),
                  _full_spec(w_atom_t.shape)],
        out_specs=pl.BlockSpec((bn, d), lambda i: (i, 0)),
        out_shape=jax.ShapeDtypeStruct((n, d), jnp.float32),
    )(f_atoms, w_atom_t)


def _edge_mlp(f_bonds, w1e, b1e, w2e, b2e, be=2000):
    e, k = f_bonds.shape
    d = w2e.shape[1]
    return pl.pallas_call(
        _edge_body,
        grid=(e // be,),
        in_specs=[pl.BlockSpec((be, k), lambda i: (i, 0)),
                  _full_spec(w1e.shape), _full_spec(b1e.shape),
                  _full_spec(w2e.shape), _full_spec(b2e.shape)],
        out_specs=pl.BlockSpec((be, d), lambda i: (i, 0)),
        out_shape=jax.ShapeDtypeStruct((e, d), jnp.float32),
    )(f_bonds, w1e, b1e, w2e, b2e)


def _node_update(p0, p1, h, w1c, b1e, w2e, b2e, w1t, b1m, w2t, b2m, g, bb,
                 relu_out, bn=2000):
    n, d = h.shape
    body = functools.partial(_node_body, relu_out=relu_out)
    row = pl.BlockSpec((bn, d), lambda i: (i, 0))
    return pl.pallas_call(
        body,
        grid=(n // bn,),
        in_specs=[row, row, row,
                  _full_spec(w1c.shape), _full_spec(b1e.shape),
                  _full_spec(w2e.shape), _full_spec(b2e.shape),
                  _full_spec(w1t.shape), _full_spec(b1m.shape),
                  _full_spec(w2t.shape), _full_spec(b2m.shape),
                  _full_spec(g.shape), _full_spec(bb.shape)],
        out_specs=row,
        out_shape=jax.ShapeDtypeStruct((n, d), jnp.float32),
    )(p0, p1, h, w1c, b1e, w2e, b2e, w1t, b1m, w2t, b2m, g, bb)


def _make_sc_aggregate(n, e, d, npad):
    c = _CHUNK
    nw = _NC * _NS
    nchunks = e // c
    cpw = nchunks // nw
    assert cpw * nw == nchunks and cpw % 2 == 1
    rps = npad // _NS
    nbuf = 2
    mesh = plsc.VectorSubcoreMesh(core_axis_name="c", subcore_axis_name="s")

    scratch = [
        pltpu.VMEM((nbuf, 2, c), jnp.int32),
        pltpu.VMEM((nbuf, c, d), jnp.float32),
        pltpu.VMEM((nbuf, c, d), jnp.float32),
        pltpu.VMEM_SHARED((npad, d), jnp.float32),
        pltpu.SemaphoreType.DMA((nbuf,)),
        pltpu.SemaphoreType.DMA((nbuf,)),
        pltpu.SemaphoreType.DMA((nbuf,)),
    ]

    @functools.partial(
        pl.kernel,
        mesh=mesh,
        out_type=jax.ShapeDtypeStruct((_NC, npad, d), jnp.float32),
        scratch_types=scratch,
    )
    def sc(h_hbm, ee_hbm, idx2_hbm, z_hbm, out_hbm,
           sd_v, hrows_v, ee_v, acc, sem_g, sem_e, sem_s):
        cid = lax.axis_index("c")
        sid = lax.axis_index("s")
        wid = cid * _NS + sid

        pltpu.sync_copy(z_hbm.at[pl.ds(sid * rps, rps)],
                        acc.at[pl.ds(sid * rps, rps)])
        plsc.subcore_barrier()

        def gather_desc(b):
            return pltpu.make_async_copy(
                h_hbm.at[sd_v.at[b, 0]], hrows_v.at[b], sem_g.at[b])

        def ee_desc(b, ck):
            return pltpu.make_async_copy(
                ee_hbm.at[pl.ds(ck * c, c)], ee_v.at[b], sem_e.at[b])

        def scatter_descs(b):
            return (
                pltpu.make_async_copy(hrows_v.at[b],
                                      acc.at[sd_v.at[b, 1]], sem_s.at[b]),
                pltpu.make_async_copy(ee_v.at[b],
                                      acc.at[sd_v.at[b, 1]], sem_s.at[b]),
            )

        def fill(t, b):
            ck = wid + nw * t
            pltpu.sync_copy(idx2_hbm.at[ck], sd_v.at[b])
            gather_desc(b).start()
            ee_desc(b, ck).start()

        def drain(b):
            gather_desc(b).wait()
            ee_desc(b, 0).wait()
            d0, d1 = scatter_descs(b)
            d0.start(add=True)
            d1.start(add=True)

        def wait_sc(b):
            d0, d1 = scatter_descs(b)
            d0.wait()
            d1.wait()

        fill(0, 0)
        fill(1, 1)
        drain(0)

        def group(tt, carry):
            for b in range(nbuf):
                t = nbuf * tt + b
                wait_sc(b)
                fill(t, b)
                drain(1 - b)
            return carry

        lax.fori_loop(1, (cpw - 1) // nbuf, group, 0)
        wait_sc(0)
        fill(cpw - 1, 0)
        drain(1)
        drain(0)
        wait_sc(1)
        wait_sc(0)

        plsc.subcore_barrier()
        pltpu.sync_copy(acc.at[pl.ds(sid * rps, rps)],
                        out_hbm.at[cid].at[pl.ds(sid * rps, rps)])

    return sc


def _aggregate_partials(h, ee, idx2, zeros_nd):
    n, d = h.shape
    e = idx2.shape[0] * idx2.shape[2]
    npad = zeros_nd.shape[0]
    return _make_sc_aggregate(n, e, d, npad)(h, ee, idx2, zeros_nd)


def _make_sc_to(n_slots, e, maxnb):
    nw = _NC * _NS
    ids_pw = e // nw
    chunk = 2000
    nchunk = n_slots // chunk
    tail = n_slots - nchunk * chunk
    mesh = plsc.VectorSubcoreMesh(core_axis_name="c", subcore_axis_name="s")

    @functools.partial(
        pl.kernel,
        mesh=mesh,
        out_type=jax.ShapeDtypeStruct((e,), jnp.int32),
        scratch_types=[
            pltpu.VMEM((chunk,), jnp.int32),
            pltpu.VMEM((ids_pw,), jnp.int32),
        ],
        compiler_params=pltpu.CompilerParams(needs_layout_passes=False),
    )
    def sc(a2b_hbm, zi_hbm, to_hbm, buf_v, tbl_v):
        cid = lax.axis_index("c")
        sid = lax.axis_index("s")
        wid = cid * _NS + sid
        lo = wid * ids_pw
        lo1 = jnp.maximum(lo, 1)
        hi = lo + ids_pw
        lane = lax.iota(jnp.int32, 16)
        ltrue = lane >= jnp.minimum(sid, 0)

        pltpu.sync_copy(zi_hbm, tbl_v)

        def scan_block(p0, nvec):
            def vbody(k, carry):
                off = k * 16 + lane
                v = plsc.load_gather(buf_v, [off], mask=ltrue)
                mask = (v >= lo1) & (v < hi)
                _, last = plsc.scan_count(v, mask)
                row = lax.div(p0 + off, jnp.int32(maxnb))
                plsc.store_scatter(tbl_v, [v - lo], row, mask=last)
                return carry
            lax.fori_loop(0, nvec, vbody, 0)

        def cbody(t, carry):
            pltpu.sync_copy(a2b_hbm.at[pl.ds(t * chunk, chunk)], buf_v)
            scan_block(t * chunk, chunk // 16)
            return carry
        lax.fori_loop(0, nchunk, cbody, 0)

        if tail:
            pltpu.sync_copy(a2b_hbm.at[pl.ds(nchunk * chunk, tail)],
                            buf_v.at[pl.ds(0, tail)])
            scan_block(nchunk * chunk, tail // 16)

        pltpu.sync_copy(tbl_v, to_hbm.at[pl.ds(lo, ids_pw)])

    return sc


def kernel(f_atoms, f_bonds, a2b, b2a, b2revb, undirected_b2a, w_atom,
           mlp_W1, mlp_b1, mlp_W2, mlp_b2, wb_W1, wb_b1, wb_W2, wb_b2,
           bn_g, bn_b):
    n, d = f_atoms.shape
    e = b2a.shape[0]
    bf1 = f_bonds.shape[1]
    depth = mlp_W1.shape[0]

    maxnb = a2b.shape[1]
    a2b_flat = a2b.reshape(-1)
    zi = jnp.zeros((e // (_NC * _NS),), jnp.int32)
    dst = _make_sc_to(a2b_flat.shape[0], e, maxnb)(a2b_flat, zi)
    src = b2a
    idx2 = jnp.stack([src.reshape(-1, _CHUNK), dst.reshape(-1, _CHUNK)],
                     axis=1)

    npad = ((n + 8 * _NS - 1) // (8 * _NS)) * (8 * _NS)
    zeros_nd = jnp.zeros((npad, d), jnp.float32)

    h = _h0(f_atoms, w_atom.T)

    for l in range(depth):
        w1e = wb_W1[l][:, :bf1].T
        b1e = wb_b1[l].reshape(1, d)
        w2e = wb_W2[l].T
        b2e = wb_b2[l].reshape(1, d)
        w1c = wb_W1[l][:, bf1].reshape(1, d)

        ee = _edge_mlp(f_bonds, w1e, b1e, w2e, b2e)
        partials = _aggregate_partials(h, ee, idx2, zeros_nd)

        w1t = mlp_W1[l].T
        b1m = mlp_b1[l].reshape(1, -1)
        w2t = mlp_W2[l].T
        b2m = mlp_b2[l].reshape(1, d)
        g = bn_g[l].reshape(1, d)
        bb = bn_b[l].reshape(1, d)

        h = _node_update(partials[0], partials[1], h,
                         w1c, b1e, w2e, b2e,
                         w1t, b1m, w2t, b2m, g, bb,
                         relu_out=(l < depth - 1))

    return h


# speedup vs baseline: 6.8239x; 1.0383x over previous
import functools

import jax
import jax.numpy as jnp
from jax import lax
from jax.experimental import pallas as pl
from jax.experimental.pallas import tpu as pltpu
from jax.experimental.pallas import tpu_sc as plsc

_EPS = 1e-5
_NC = 2
_NS = 16
_CHUNK = 80


def _h0_body(x_ref, wt_ref, o_ref):
    o_ref[...] = jnp.dot(x_ref[...], wt_ref[...],
                         preferred_element_type=jnp.float32)


def _edge_body(fb_ref, w1_ref, b1_ref, w2_ref, b2_ref, o_ref):
    t = jnp.dot(fb_ref[...], w1_ref[...],
                preferred_element_type=jnp.float32) + b1_ref[...]
    t = jnp.maximum(t, 0.0)
    o_ref[...] = jnp.dot(t, w2_ref[...],
                         preferred_element_type=jnp.float32) + b2_ref[...]


def _node_body(p0_ref, p1_ref, h_ref, w1c_ref, b1e_ref, w2e_ref, b2e_ref,
               w1t_ref, b1m_ref, w2t_ref, b2m_ref, g_ref, bb_ref, o_ref,
               *, relu_out):
    ee_loop = jnp.maximum(w1c_ref[...] + b1e_ref[...], 0.0)
    ee_loop = jnp.dot(ee_loop, w2e_ref[...],
                      preferred_element_type=jnp.float32) + b2e_ref[...]
    aggr = p0_ref[...] + p1_ref[...] + h_ref[...] + ee_loop
    t = jnp.dot(aggr, w1t_ref[...],
                preferred_element_type=jnp.float32) + b1m_ref[...]
    t = jnp.maximum(t, 0.0)
    hh = jnp.dot(t, w2t_ref[...],
                 preferred_element_type=jnp.float32) + b2m_ref[...]
    scale = 1.0 / jnp.sqrt(1.0 + _EPS)
    hh = hh * (g_ref[...] * scale) + bb_ref[...]
    if relu_out:
        hh = jnp.maximum(hh, 0.0)
    o_ref[...] = hh


def _full_spec(shape):
    nd = len(shape)
    return pl.BlockSpec(shape, lambda i: (0,) * nd)


def _h0(f_atoms, w_atom_t, bn=2000):
    n, d = f_atoms.shape
    return pl.pallas_call(
        _h0_body,
        grid=(n // bn,),
        in_specs=[pl.BlockSpec((bn, d), lambda i: (i, 0)),
                  _full_spec(w_atom_t.shape)],
        out_specs=pl.BlockSpec((bn, d), lambda i: (i, 0)),
        out_shape=jax.ShapeDtypeStruct((n, d), jnp.float32),
    )(f_atoms, w_atom_t)


def _edge_mlp(f_bonds, w1e, b1e, w2e, b2e, be=2000):
    e, k = f_bonds.shape
    d = w2e.shape[1]
    return pl.pallas_call(
        _edge_body,
        grid=(e // be,),
        in_specs=[pl.BlockSpec((be, k), lambda i: (i, 0)),
                  _full_spec(w1e.shape), _full_spec(b1e.shape),
                  _full_spec(w2e.shape), _full_spec(b2e.shape)],
        out_specs=pl.BlockSpec((be, d), lambda i: (i, 0)),
        out_shape=jax.ShapeDtypeStruct((e, d), jnp.float32),
    )(f_bonds, w1e, b1e, w2e, b2e)


def _node_update(p0, p1, h, w1c, b1e, w2e, b2e, w1t, b1m, w2t, b2m, g, bb,
                 relu_out, bn=2000):
    n, d = h.shape
    body = functools.partial(_node_body, relu_out=relu_out)
    row = pl.BlockSpec((bn, d), lambda i: (i, 0))
    return pl.pallas_call(
        body,
        grid=(n // bn,),
        in_specs=[row, row, row,
                  _full_spec(w1c.shape), _full_spec(b1e.shape),
                  _full_spec(w2e.shape), _full_spec(b2e.shape),
                  _full_spec(w1t.shape), _full_spec(b1m.shape),
                  _full_spec(w2t.shape), _full_spec(b2m.shape),
                  _full_spec(g.shape), _full_spec(bb.shape)],
        out_specs=row,
        out_shape=jax.ShapeDtypeStruct((n, d), jnp.float32),
    )(p0, p1, h, w1c, b1e, w2e, b2e, w1t, b1m, w2t, b2m, g, bb)


def _make_sc_aggregate(n, e, d, npad):
    c = _CHUNK
    nw = _NC * _NS
    nchunks = e // c
    cpw = nchunks // nw
    assert cpw * nw == nchunks and cpw % 2 == 1
    rps = npad // _NS
    nbuf = 2
    mesh = plsc.VectorSubcoreMesh(core_axis_name="c", subcore_axis_name="s")

    scratch = [
        pltpu.VMEM((nbuf, 2, c), jnp.int32),
        pltpu.VMEM((nbuf, c, d), jnp.float32),
        pltpu.VMEM((nbuf, c, d), jnp.float32),
        pltpu.VMEM_SHARED((npad, d), jnp.float32),
        pltpu.SemaphoreType.DMA((nbuf,)),
        pltpu.SemaphoreType.DMA((nbuf,)),
        pltpu.SemaphoreType.DMA((nbuf,)),
    ]

    @functools.partial(
        pl.kernel,
        mesh=mesh,
        out_type=jax.ShapeDtypeStruct((_NC, npad, d), jnp.float32),
        scratch_types=scratch,
    )
    def sc(h_hbm, ee_hbm, idx2_hbm, z_hbm, out_hbm,
           sd_v, hrows_v, ee_v, acc, sem_g, sem_e, sem_s):
        cid = lax.axis_index("c")
        sid = lax.axis_index("s")
        wid = cid * _NS + sid

        pltpu.sync_copy(z_hbm.at[pl.ds(sid * rps, rps)],
                        acc.at[pl.ds(sid * rps, rps)])
        plsc.subcore_barrier()

        def gather_desc(b):
            return pltpu.make_async_copy(
                h_hbm.at[sd_v.at[b, 0]], hrows_v.at[b], sem_g.at[b])

        def ee_desc(b, ck):
            return pltpu.make_async_copy(
                ee_hbm.at[pl.ds(ck * c, c)], ee_v.at[b], sem_e.at[b])

        def scatter_descs(b):
            return (
                pltpu.make_async_copy(hrows_v.at[b],
                                      acc.at[sd_v.at[b, 1]], sem_s.at[b]),
                pltpu.make_async_copy(ee_v.at[b],
                                      acc.at[sd_v.at[b, 1]], sem_s.at[b]),
            )

        def fill(t, b):
            ck = wid + nw * t
            pltpu.sync_copy(idx2_hbm.at[ck], sd_v.at[b])
            gather_desc(b).start()
            ee_desc(b, ck).start()

        def drain(b):
            gather_desc(b).wait()
            ee_desc(b, 0).wait()
            d0, d1 = scatter_descs(b)
            d0.start(add=True)
            d1.start(add=True)

        def wait_sc(b):
            d0, d1 = scatter_descs(b)
            d0.wait()
            d1.wait()

        fill(0, 0)
        fill(1, 1)
        drain(0)

        def group(tt, carry):
            for b in range(nbuf):
                t = nbuf * tt + b
                wait_sc(b)
                fill(t, b)
                drain(1 - b)
            return carry

        lax.fori_loop(1, (cpw - 1) // nbuf, group, 0)
        wait_sc(0)
        fill(cpw - 1, 0)
        drain(1)
        drain(0)
        wait_sc(1)
        wait_sc(0)

        plsc.subcore_barrier()
        pltpu.sync_copy(acc.at[pl.ds(sid * rps, rps)],
                        out_hbm.at[cid].at[pl.ds(sid * rps, rps)])

    return sc


def _aggregate_partials(h, ee, idx2, zeros_nd):
    n, d = h.shape
    e = idx2.shape[0] * idx2.shape[2]
    npad = zeros_nd.shape[0]
    return _make_sc_aggregate(n, e, d, npad)(h, ee, idx2, zeros_nd)


def _make_sc_to(n_slots, e, maxnb):
    nw = _NC * _NS
    ids_pw = e // nw
    chunk = 2000
    nchunk = n_slots // chunk
    tail = n_slots - nchunk * chunk
    mesh = plsc.VectorSubcoreMesh(core_axis_name="c", subcore_axis_name="s")

    @functools.partial(
        pl.kernel,
        mesh=mesh,
        out_type=jax.ShapeDtypeStruct((e,), jnp.int32),
        scratch_types=[
            pltpu.VMEM((2, chunk), jnp.int32),
            pltpu.VMEM((ids_pw,), jnp.int32),
            pltpu.SemaphoreType.DMA((2,)),
        ],
        compiler_params=pltpu.CompilerParams(needs_layout_passes=False),
    )
    def sc(a2b_hbm, zi_hbm, to_hbm, buf_v, tbl_v, sem):
        cid = lax.axis_index("c")
        sid = lax.axis_index("s")
        wid = cid * _NS + sid
        lo = wid * ids_pw
        lo1 = jnp.maximum(lo, 1)
        hi = lo + ids_pw
        lane = lax.iota(jnp.int32, 16)

        pltpu.sync_copy(zi_hbm, tbl_v)

        def load_desc(t, b):
            return pltpu.make_async_copy(a2b_hbm.at[t], buf_v.at[b],
                                         sem.at[b])

        def scan_block(p0, b):
            for k in range(chunk // 16):
                v = buf_v[b, pl.ds(k * 16, 16)]
                mask = (v >= lo1) & (v < hi)
                _, last = plsc.scan_count(v, mask)
                row = lax.div(p0 + k * 16 + lane, jnp.int32(maxnb))
                plsc.store_scatter(tbl_v, [v - lo], row, mask=last)

        load_desc(0, 0).start()

        def cbody(tt, carry):
            for b in range(2):
                t = 2 * tt + b
                load_desc(t, b).wait()

                @pl.when(t + 1 < nchunk)
                def _():
                    load_desc(t + 1, 1 - b).start()

                scan_block(t * chunk, b)
            return carry
        lax.fori_loop(0, nchunk // 2, cbody, 0)

        pltpu.sync_copy(tbl_v, to_hbm.at[pl.ds(lo, ids_pw)])

    def call(a2b_flat, zi):
        return sc(a2b_flat.reshape(nchunk, chunk), zi)

    return call


def kernel(f_atoms, f_bonds, a2b, b2a, b2revb, undirected_b2a, w_atom,
           mlp_W1, mlp_b1, mlp_W2, mlp_b2, wb_W1, wb_b1, wb_W2, wb_b2,
           bn_g, bn_b):
    n, d = f_atoms.shape
    e = b2a.shape[0]
    bf1 = f_bonds.shape[1]
    depth = mlp_W1.shape[0]

    maxnb = a2b.shape[1]
    a2b_flat = a2b.reshape(-1)
    zi = jnp.zeros((e // (_NC * _NS),), jnp.int32)
    dst = _make_sc_to(a2b_flat.shape[0], e, maxnb)(a2b_flat, zi)
    src = b2a
    idx2 = jnp.stack([src.reshape(-1, _CHUNK), dst.reshape(-1, _CHUNK)],
                     axis=1)

    npad = ((n + 8 * _NS - 1) // (8 * _NS)) * (8 * _NS)
    zeros_nd = jnp.zeros((npad, d), jnp.float32)

    h = _h0(f_atoms, w_atom.T)

    for l in range(depth):
        w1e = wb_W1[l][:, :bf1].T
        b1e = wb_b1[l].reshape(1, d)
        w2e = wb_W2[l].T
        b2e = wb_b2[l].reshape(1, d)
        w1c = wb_W1[l][:, bf1].reshape(1, d)

        ee = _edge_mlp(f_bonds, w1e, b1e, w2e, b2e)
        partials = _aggregate_partials(h, ee, idx2, zeros_nd)

        w1t = mlp_W1[l].T
        b1m = mlp_b1[l].reshape(1, -1)
        w2t = mlp_W2[l].T
        b2m = mlp_b2[l].reshape(1, d)
        g = bn_g[l].reshape(1, d)
        bb = bn_b[l].reshape(1, d)

        h = _node_update(partials[0], partials[1], h,
                         w1c, b1e, w2e, b2e,
                         w1t, b1m, w2t, b2m, g, bb,
                         relu_out=(l < depth - 1))

    return h


# speedup vs baseline: 7.4291x; 1.0887x over previous
import functools

import jax
import jax.numpy as jnp
from jax import lax
from jax.experimental import pallas as pl
from jax.experimental.pallas import tpu as pltpu
from jax.experimental.pallas import tpu_sc as plsc

_EPS = 1e-5
_NC = 2
_NS = 16
_CHUNK = 80


def _h0_body(x_ref, wt_ref, o_ref):
    o_ref[...] = jnp.dot(x_ref[...], wt_ref[...],
                         preferred_element_type=jnp.float32)


def _edge_body(fb_ref, w1_ref, b1_ref, w2_ref, b2_ref, o_ref):
    t = jnp.dot(fb_ref[...], w1_ref[...],
                preferred_element_type=jnp.float32) + b1_ref[...]
    t = jnp.maximum(t, 0.0)
    o_ref[...] = jnp.dot(t, w2_ref[...],
                         preferred_element_type=jnp.float32) + b2_ref[...]


def _node_body(p0_ref, p1_ref, h_ref, w1c_ref, b1e_ref, w2e_ref, b2e_ref,
               w1t_ref, b1m_ref, w2t_ref, b2m_ref, g_ref, bb_ref, o_ref,
               *, relu_out):
    ee_loop = jnp.maximum(w1c_ref[...] + b1e_ref[...], 0.0)
    ee_loop = jnp.dot(ee_loop, w2e_ref[...],
                      preferred_element_type=jnp.float32) + b2e_ref[...]
    aggr = p0_ref[...] + p1_ref[...] + h_ref[...] + ee_loop
    t = jnp.dot(aggr, w1t_ref[...],
                preferred_element_type=jnp.float32) + b1m_ref[...]
    t = jnp.maximum(t, 0.0)
    hh = jnp.dot(t, w2t_ref[...],
                 preferred_element_type=jnp.float32) + b2m_ref[...]
    scale = 1.0 / jnp.sqrt(1.0 + _EPS)
    hh = hh * (g_ref[...] * scale) + bb_ref[...]
    if relu_out:
        hh = jnp.maximum(hh, 0.0)
    o_ref[...] = hh


def _full_spec(shape):
    nd = len(shape)
    return pl.BlockSpec(shape, lambda i: (0,) * nd)


def _h0(f_atoms, w_atom_t, bn=2000):
    n, d = f_atoms.shape
    return pl.pallas_call(
        _h0_body,
        grid=(n // bn,),
        in_specs=[pl.BlockSpec((bn, d), lambda i: (i, 0)),
                  _full_spec(w_atom_t.shape)],
        out_specs=pl.BlockSpec((bn, d), lambda i: (i, 0)),
        out_shape=jax.ShapeDtypeStruct((n, d), jnp.float32),
    )(f_atoms, w_atom_t)


def _edge_mlp(f_bonds, w1e, b1e, w2e, b2e, be=2000):
    e, k = f_bonds.shape
    d = w2e.shape[1]
    return pl.pallas_call(
        _edge_body,
        grid=(e // be,),
        in_specs=[pl.BlockSpec((be, k), lambda i: (i, 0)),
                  _full_spec(w1e.shape), _full_spec(b1e.shape),
                  _full_spec(w2e.shape), _full_spec(b2e.shape)],
        out_specs=pl.BlockSpec((be, d), lambda i: (i, 0)),
        out_shape=jax.ShapeDtypeStruct((e, d), jnp.float32),
    )(f_bonds, w1e, b1e, w2e, b2e)


def _node_update(p0, p1, h, w1c, b1e, w2e, b2e, w1t, b1m, w2t, b2m, g, bb,
                 relu_out, bn=2000):
    n, d = h.shape
    body = functools.partial(_node_body, relu_out=relu_out)
    row = pl.BlockSpec((bn, d), lambda i: (i, 0))
    return pl.pallas_call(
        body,
        grid=(n // bn,),
        in_specs=[row, row, row,
                  _full_spec(w1c.shape), _full_spec(b1e.shape),
                  _full_spec(w2e.shape), _full_spec(b2e.shape),
                  _full_spec(w1t.shape), _full_spec(b1m.shape),
                  _full_spec(w2t.shape), _full_spec(b2m.shape),
                  _full_spec(g.shape), _full_spec(bb.shape)],
        out_specs=row,
        out_shape=jax.ShapeDtypeStruct((n, d), jnp.float32),
    )(p0, p1, h, w1c, b1e, w2e, b2e, w1t, b1m, w2t, b2m, g, bb)


def _make_sc_aggregate(n, e, d, npad):
    c = _CHUNK
    nw = _NC * _NS
    nchunks = e // c
    cpw = nchunks // nw
    assert cpw * nw == nchunks and cpw % 2 == 1
    rps = npad // _NS
    nbuf = 2
    mesh = plsc.VectorSubcoreMesh(core_axis_name="c", subcore_axis_name="s")

    scratch = [
        pltpu.VMEM((nbuf, 2, c), jnp.int32),
        pltpu.VMEM((nbuf, c, d), jnp.float32),
        pltpu.VMEM((nbuf, c, d), jnp.float32),
        pltpu.VMEM_SHARED((npad, d), jnp.float32),
        pltpu.SemaphoreType.DMA((nbuf,)),
        pltpu.SemaphoreType.DMA((nbuf,)),
        pltpu.SemaphoreType.DMA((nbuf,)),
    ]

    @functools.partial(
        pl.kernel,
        mesh=mesh,
        out_type=jax.ShapeDtypeStruct((_NC, npad, d), jnp.float32),
        scratch_types=scratch,
    )
    def sc(h_hbm, ee_hbm, idx2_hbm, z_hbm, out_hbm,
           sd_v, hrows_v, ee_v, acc, sem_g, sem_e, sem_s):
        cid = lax.axis_index("c")
        sid = lax.axis_index("s")
        wid = cid * _NS + sid

        pltpu.sync_copy(z_hbm.at[pl.ds(sid * rps, rps)],
                        acc.at[pl.ds(sid * rps, rps)])
        plsc.subcore_barrier()

        def gather_desc(b):
            return pltpu.make_async_copy(
                h_hbm.at[sd_v.at[b, 0]], hrows_v.at[b], sem_g.at[b])

        def ee_desc(b, ck):
            return pltpu.make_async_copy(
                ee_hbm.at[pl.ds(ck * c, c)], ee_v.at[b], sem_e.at[b])

        def scatter_descs(b):
            return (
                pltpu.make_async_copy(hrows_v.at[b],
                                      acc.at[sd_v.at[b, 1]], sem_s.at[b]),
                pltpu.make_async_copy(ee_v.at[b],
                                      acc.at[sd_v.at[b, 1]], sem_s.at[b]),
            )

        def fill(t, b):
            ck = wid + nw * t
            pltpu.sync_copy(idx2_hbm.at[ck], sd_v.at[b])
            gather_desc(b).start()
            ee_desc(b, ck).start()

        def drain(b):
            gather_desc(b).wait()
            ee_desc(b, 0).wait()
            d0, d1 = scatter_descs(b)
            d0.start(add=True)
            d1.start(add=True)

        def wait_sc(b):
            d0, d1 = scatter_descs(b)
            d0.wait()
            d1.wait()

        fill(0, 0)
        fill(1, 1)
        drain(0)

        def group(tt, carry):
            for b in range(nbuf):
                t = nbuf * tt + b
                wait_sc(b)
                fill(t, b)
                drain(1 - b)
            return carry

        lax.fori_loop(1, (cpw - 1) // nbuf, group, 0)
        wait_sc(0)
        fill(cpw - 1, 0)
        drain(1)
        drain(0)
        wait_sc(1)
        wait_sc(0)

        plsc.subcore_barrier()
        pltpu.sync_copy(acc.at[pl.ds(sid * rps, rps)],
                        out_hbm.at[cid].at[pl.ds(sid * rps, rps)])

    return sc


def _aggregate_partials(h, ee, idx2, zeros_nd):
    n, d = h.shape
    e = idx2.shape[0] * idx2.shape[2]
    npad = zeros_nd.shape[0]
    return _make_sc_aggregate(n, e, d, npad)(h, ee, idx2, zeros_nd)


def _make_sc_to(n_slots, e, maxnb):
    nw = _NC * _NS
    ids_pw = e // nw
    chunk = 2000
    nchunk = n_slots // chunk
    tail = n_slots - nchunk * chunk
    mesh = plsc.VectorSubcoreMesh(core_axis_name="c", subcore_axis_name="s")

    @functools.partial(
        pl.kernel,
        mesh=mesh,
        out_type=jax.ShapeDtypeStruct((e,), jnp.int32),
        scratch_types=[
            pltpu.VMEM((chunk,), jnp.int32),
            pltpu.VMEM((chunk,), jnp.int32),
            pltpu.VMEM((ids_pw,), jnp.int32),
            pltpu.SemaphoreType.DMA((2,)),
        ],
        compiler_params=pltpu.CompilerParams(needs_layout_passes=False),
    )
    def sc(a2b_hbm, zi_hbm, to_hbm, buf0_v, buf1_v, tbl_v, sem):
        bufs = (buf0_v, buf1_v)
        cid = lax.axis_index("c")
        sid = lax.axis_index("s")
        wid = cid * _NS + sid
        lo = wid * ids_pw
        lo1 = jnp.maximum(lo, 1)
        hi = lo + ids_pw
        lane = lax.iota(jnp.int32, 16)
        ltrue = lane >= jnp.minimum(sid, 0)

        pltpu.sync_copy(zi_hbm, tbl_v)

        def load_desc(t, b):
            return pltpu.make_async_copy(
                a2b_hbm.at[pl.ds(t * chunk, chunk)], bufs[b], sem.at[b])

        def scan_block(p0, b):
            def vbody(k, carry):
                off = k * 16 + lane
                v = plsc.load_gather(bufs[b], [off], mask=ltrue)
                mask = (v >= lo1) & (v < hi)
                _, last = plsc.scan_count(v, mask)
                row = lax.div(p0 + off, jnp.int32(maxnb))
                plsc.store_scatter(tbl_v, [v - lo], row, mask=last)
                return carry
            lax.fori_loop(0, chunk // 16, vbody, 0)

        load_desc(0, 0).start()

        def cbody(tt, carry):
            for b in range(2):
                t = 2 * tt + b
                load_desc(t, b).wait()

                @pl.when(t + 1 < nchunk)
                def _():
                    load_desc(t + 1, 1 - b).start()

                scan_block(t * chunk, b)
            return carry
        lax.fori_loop(0, nchunk // 2, cbody, 0)

        pltpu.sync_copy(tbl_v, to_hbm.at[pl.ds(lo, ids_pw)])

    return sc


def kernel(f_atoms, f_bonds, a2b, b2a, b2revb, undirected_b2a, w_atom,
           mlp_W1, mlp_b1, mlp_W2, mlp_b2, wb_W1, wb_b1, wb_W2, wb_b2,
           bn_g, bn_b):
    n, d = f_atoms.shape
    e = b2a.shape[0]
    bf1 = f_bonds.shape[1]
    depth = mlp_W1.shape[0]

    maxnb = a2b.shape[1]
    a2b_flat = a2b.reshape(-1)
    zi = jnp.zeros((e // (_NC * _NS),), jnp.int32)
    dst = _make_sc_to(a2b_flat.shape[0], e, maxnb)(a2b_flat, zi)
    src = b2a
    idx2 = jnp.stack([src.reshape(-1, _CHUNK), dst.reshape(-1, _CHUNK)],
                     axis=1)

    npad = ((n + 8 * _NS - 1) // (8 * _NS)) * (8 * _NS)
    zeros_nd = jnp.zeros((npad, d), jnp.float32)

    h = _h0(f_atoms, w_atom.T)

    for l in range(depth):
        w1e = wb_W1[l][:, :bf1].T
        b1e = wb_b1[l].reshape(1, d)
        w2e = wb_W2[l].T
        b2e = wb_b2[l].reshape(1, d)
        w1c = wb_W1[l][:, bf1].reshape(1, d)

        ee = _edge_mlp(f_bonds, w1e, b1e, w2e, b2e)
        partials = _aggregate_partials(h, ee, idx2, zeros_nd)

        w1t = mlp_W1[l].T
        b1m = mlp_b1[l].reshape(1, -1)
        w2t = mlp_W2[l].T
        b2m = mlp_b2[l].reshape(1, d)
        g = bn_g[l].reshape(1, d)
        bb = bn_b[l].reshape(1, d)

        h = _node_update(partials[0], partials[1], h,
                         w1c, b1e, w2e, b2e,
                         w1t, b1m, w2t, b2m, g, bb,
                         relu_out=(l < depth - 1))

    return h


# speedup vs baseline: 7.4764x; 1.0064x over previous
import functools

import jax
import jax.numpy as jnp
from jax import lax
from jax.experimental import pallas as pl
from jax.experimental.pallas import tpu as pltpu
from jax.experimental.pallas import tpu_sc as plsc

_EPS = 1e-5
_NC = 2
_NS = 16
_CHUNK = 80


def _h0_body(x_ref, wt_ref, o_ref):
    o_ref[...] = jnp.dot(x_ref[...], wt_ref[...],
                         preferred_element_type=jnp.float32)


def _edge_body(fb_ref, w1_ref, b1_ref, w2_ref, b2_ref, o_ref):
    t = jnp.dot(fb_ref[...], w1_ref[...],
                preferred_element_type=jnp.float32) + b1_ref[...]
    t = jnp.maximum(t, 0.0)
    o_ref[...] = jnp.dot(t, w2_ref[...],
                         preferred_element_type=jnp.float32) + b2_ref[...]


def _node_body(p0_ref, p1_ref, h_ref, w1c_ref, b1e_ref, w2e_ref, b2e_ref,
               w1t_ref, b1m_ref, w2t_ref, b2m_ref, g_ref, bb_ref, o_ref,
               *, relu_out):
    ee_loop = jnp.maximum(w1c_ref[...] + b1e_ref[...], 0.0)
    ee_loop = jnp.dot(ee_loop, w2e_ref[...],
                      preferred_element_type=jnp.float32) + b2e_ref[...]
    aggr = p0_ref[...] + p1_ref[...] + h_ref[...] + ee_loop
    t = jnp.dot(aggr, w1t_ref[...],
                preferred_element_type=jnp.float32) + b1m_ref[...]
    t = jnp.maximum(t, 0.0)
    hh = jnp.dot(t, w2t_ref[...],
                 preferred_element_type=jnp.float32) + b2m_ref[...]
    scale = 1.0 / jnp.sqrt(1.0 + _EPS)
    hh = hh * (g_ref[...] * scale) + bb_ref[...]
    if relu_out:
        hh = jnp.maximum(hh, 0.0)
    o_ref[...] = hh


def _full_spec(shape):
    nd = len(shape)
    return pl.BlockSpec(shape, lambda i: (0,) * nd)


def _h0(f_atoms, w_atom_t, bn=2000):
    n, d = f_atoms.shape
    return pl.pallas_call(
        _h0_body,
        grid=(n // bn,),
        in_specs=[pl.BlockSpec((bn, d), lambda i: (i, 0)),
                  _full_spec(w_atom_t.shape)],
        out_specs=pl.BlockSpec((bn, d), lambda i: (i, 0)),
        out_shape=jax.ShapeDtypeStruct((n, d), jnp.float32),
    )(f_atoms, w_atom_t)


def _edge_mlp(f_bonds, w1e, b1e, w2e, b2e, be=2000):
    e, k = f_bonds.shape
    d = w2e.shape[1]
    return pl.pallas_call(
        _edge_body,
        grid=(e // be,),
        in_specs=[pl.BlockSpec((be, k), lambda i: (i, 0)),
                  _full_spec(w1e.shape), _full_spec(b1e.shape),
                  _full_spec(w2e.shape), _full_spec(b2e.shape)],
        out_specs=pl.BlockSpec((be, d), lambda i: (i, 0)),
        out_shape=jax.ShapeDtypeStruct((e, d), jnp.float32),
    )(f_bonds, w1e, b1e, w2e, b2e)


def _node_update(p0, p1, h, w1c, b1e, w2e, b2e, w1t, b1m, w2t, b2m, g, bb,
                 relu_out, bn=2000):
    n, d = h.shape
    body = functools.partial(_node_body, relu_out=relu_out)
    row = pl.BlockSpec((bn, d), lambda i: (i, 0))
    return pl.pallas_call(
        body,
        grid=(n // bn,),
        in_specs=[row, row, row,
                  _full_spec(w1c.shape), _full_spec(b1e.shape),
                  _full_spec(w2e.shape), _full_spec(b2e.shape),
                  _full_spec(w1t.shape), _full_spec(b1m.shape),
                  _full_spec(w2t.shape), _full_spec(b2m.shape),
                  _full_spec(g.shape), _full_spec(bb.shape)],
        out_specs=row,
        out_shape=jax.ShapeDtypeStruct((n, d), jnp.float32),
    )(p0, p1, h, w1c, b1e, w2e, b2e, w1t, b1m, w2t, b2m, g, bb)


def _make_sc_aggregate(n, e, d, npad):
    c = _CHUNK
    nw = _NC * _NS
    nchunks = e // c
    cpw = nchunks // nw
    assert cpw * nw == nchunks and cpw % 2 == 1
    rps = npad // _NS
    nbuf = 2
    mesh = plsc.VectorSubcoreMesh(core_axis_name="c", subcore_axis_name="s")

    scratch = [
        pltpu.VMEM((nbuf, 2, c), jnp.int32),
        pltpu.VMEM((nbuf, c, d), jnp.float32),
        pltpu.VMEM((nbuf, c, d), jnp.float32),
        pltpu.VMEM_SHARED((npad, d), jnp.float32),
        pltpu.SemaphoreType.DMA((nbuf,)),
        pltpu.SemaphoreType.DMA((nbuf,)),
        pltpu.SemaphoreType.DMA((nbuf,)),
    ]

    @functools.partial(
        pl.kernel,
        mesh=mesh,
        out_type=jax.ShapeDtypeStruct((_NC, npad, d), jnp.float32),
        scratch_types=scratch,
    )
    def sc(h_hbm, ee_hbm, idx2_hbm, z_hbm, out_hbm,
           sd_v, hrows_v, ee_v, acc, sem_g, sem_e, sem_s):
        cid = lax.axis_index("c")
        sid = lax.axis_index("s")
        wid = cid * _NS + sid

        pltpu.sync_copy(z_hbm.at[pl.ds(sid * rps, rps)],
                        acc.at[pl.ds(sid * rps, rps)])
        plsc.subcore_barrier()

        def gather_desc(b):
            return pltpu.make_async_copy(
                h_hbm.at[sd_v.at[b, 0]], hrows_v.at[b], sem_g.at[b])

        def ee_desc(b, ck):
            return pltpu.make_async_copy(
                ee_hbm.at[pl.ds(ck * c, c)], ee_v.at[b], sem_e.at[b])

        def scatter_descs(b):
            return (
                pltpu.make_async_copy(hrows_v.at[b],
                                      acc.at[sd_v.at[b, 1]], sem_s.at[b]),
                pltpu.make_async_copy(ee_v.at[b],
                                      acc.at[sd_v.at[b, 1]], sem_s.at[b]),
            )

        def fill(t, b):
            ck = wid + nw * t
            pltpu.sync_copy(idx2_hbm.at[ck], sd_v.at[b])
            gather_desc(b).start()
            ee_desc(b, ck).start()

        def drain(b):
            gather_desc(b).wait()
            ee_desc(b, 0).wait()
            d0, d1 = scatter_descs(b)
            d0.start(add=True)
            d1.start(add=True)

        def wait_sc(b):
            d0, d1 = scatter_descs(b)
            d0.wait()
            d1.wait()

        fill(0, 0)
        fill(1, 1)
        drain(0)

        def group(tt, carry):
            for b in range(nbuf):
                t = nbuf * tt + b
                wait_sc(b)
                fill(t, b)
                drain(1 - b)
            return carry

        lax.fori_loop(1, (cpw - 1) // nbuf, group, 0)
        wait_sc(0)
        fill(cpw - 1, 0)
        drain(1)
        drain(0)
        wait_sc(1)
        wait_sc(0)

        plsc.subcore_barrier()
        pltpu.sync_copy(acc.at[pl.ds(sid * rps, rps)],
                        out_hbm.at[cid].at[pl.ds(sid * rps, rps)])

    return sc


def _aggregate_partials(h, ee, idx2, zeros_nd):
    n, d = h.shape
    e = idx2.shape[0] * idx2.shape[2]
    npad = zeros_nd.shape[0]
    return _make_sc_aggregate(n, e, d, npad)(h, ee, idx2, zeros_nd)


def _make_sc_to(n_slots, e, maxnb):
    nw = _NC * _NS
    ids_pw = e // nw
    chunk = 4000
    nchunk = n_slots // chunk
    tail = n_slots - nchunk * chunk
    mesh = plsc.VectorSubcoreMesh(core_axis_name="c", subcore_axis_name="s")

    @functools.partial(
        pl.kernel,
        mesh=mesh,
        out_type=jax.ShapeDtypeStruct((e,), jnp.int32),
        scratch_types=[
            pltpu.VMEM((chunk,), jnp.int32),
            pltpu.VMEM((chunk,), jnp.int32),
            pltpu.VMEM((ids_pw,), jnp.int32),
            pltpu.SemaphoreType.DMA((2,)),
        ],
        compiler_params=pltpu.CompilerParams(needs_layout_passes=False),
    )
    def sc(a2b_hbm, zi_hbm, to_hbm, buf0_v, buf1_v, tbl_v, sem):
        bufs = (buf0_v, buf1_v)
        cid = lax.axis_index("c")
        sid = lax.axis_index("s")
        wid = cid * _NS + sid
        lo = wid * ids_pw
        lo1 = jnp.maximum(lo, 1)
        hi = lo + ids_pw
        lane = lax.iota(jnp.int32, 16)
        ltrue = lane >= jnp.minimum(sid, 0)

        pltpu.sync_copy(zi_hbm, tbl_v)

        def load_desc(t, b):
            return pltpu.make_async_copy(
                a2b_hbm.at[pl.ds(t * chunk, chunk)], bufs[b], sem.at[b])

        def scan_block(p0, b):
            def vbody(k, carry):
                off = k * 16 + lane
                v = plsc.load_gather(bufs[b], [off], mask=ltrue)
                mask = (v >= lo1) & (v < hi)
                _, last = plsc.scan_count(v, mask)
                row = lax.div(p0 + off, jnp.int32(maxnb))
                plsc.store_scatter(tbl_v, [v - lo], row, mask=last)
                return carry
            lax.fori_loop(0, chunk // 16, vbody, 0)

        load_desc(0, 0).start()

        def cbody(tt, carry):
            for b in range(2):
                t = 2 * tt + b
                load_desc(t, b).wait()

                @pl.when(t + 1 < nchunk)
                def _():
                    load_desc(t + 1, 1 - b).start()

                scan_block(t * chunk, b)
            return carry
        lax.fori_loop(0, nchunk // 2, cbody, 0)

        pltpu.sync_copy(tbl_v, to_hbm.at[pl.ds(lo, ids_pw)])

    return sc


def kernel(f_atoms, f_bonds, a2b, b2a, b2revb, undirected_b2a, w_atom,
           mlp_W1, mlp_b1, mlp_W2, mlp_b2, wb_W1, wb_b1, wb_W2, wb_b2,
           bn_g, bn_b):
    n, d = f_atoms.shape
    e = b2a.shape[0]
    bf1 = f_bonds.shape[1]
    depth = mlp_W1.shape[0]

    maxnb = a2b.shape[1]
    a2b_flat = a2b.reshape(-1)
    zi = jnp.zeros((e // (_NC * _NS),), jnp.int32)
    dst = _make_sc_to(a2b_flat.shape[0], e, maxnb)(a2b_flat, zi)
    src = b2a
    idx2 = jnp.stack([src.reshape(-1, _CHUNK), dst.reshape(-1, _CHUNK)],
                     axis=1)

    npad = ((n + 8 * _NS - 1) // (8 * _NS)) * (8 * _NS)
    zeros_nd = jnp.zeros((npad, d), jnp.float32)

    h = _h0(f_atoms, w_atom.T)

    for l in range(depth):
        w1e = wb_W1[l][:, :bf1].T
        b1e = wb_b1[l].reshape(1, d)
        w2e = wb_W2[l].T
        b2e = wb_b2[l].reshape(1, d)
        w1c = wb_W1[l][:, bf1].reshape(1, d)

        ee = _edge_mlp(f_bonds, w1e, b1e, w2e, b2e)
        partials = _aggregate_partials(h, ee, idx2, zeros_nd)

        w1t = mlp_W1[l].T
        b1m = mlp_b1[l].reshape(1, -1)
        w2t = mlp_W2[l].T
        b2m = mlp_b2[l].reshape(1, d)
        g = bn_g[l].reshape(1, d)
        bb = bn_b[l].reshape(1, d)

        h = _node_update(partials[0], partials[1], h,
                         w1c, b1e, w2e, b2e,
                         w1t, b1m, w2t, b2m, g, bb,
                         relu_out=(l < depth - 1))

    return h
